# baseline (speedup 1.0000x reference)
import functools

import jax
import jax.numpy as jnp
from jax import lax
from jax.experimental import pallas as pl
from jax.experimental.pallas import tpu as pltpu
from jax.experimental.pallas import tpu_sc as plsc

N = 10000
E = 160000
D = 256
G = 50

NC = 2
NS = 16
L = 16

W = D // NC
EPT = E // NS
CH = 128
NCHUNK = 84
EPT_PAD = NCHUNK * CH
HCH = 128
HNCHUNK = 80
HEPT_PAD = HNCHUNK * HCH
ZR = 32

N_PAD = 10240
TPR = N_PAD // NS
BN = 512
NB = N_PAD // BN
GP = 64
KP = 48


def _mesh():
    return plsc.VectorSubcoreMesh(core_axis_name="c", subcore_axis_name="s",
                                  num_cores=NC, num_subcores=NS)


def _sc_hist(idx2):
    out_ty = jax.ShapeDtypeStruct((NC, N_PAD, L), jnp.float32)
    scratch = [
        pltpu.VMEM((HCH,), jnp.int32),
        pltpu.VMEM((HCH,), jnp.int32),
        pltpu.VMEM((HCH, L), jnp.float32),
        pltpu.VMEM((ZR, L), jnp.float32),
        pltpu.VMEM_SHARED((N_PAD, L), jnp.float32),
    ]

    @functools.partial(pl.kernel, out_type=out_ty, mesh=_mesh(),
                       scratch_types=scratch)
    def k(idx_hbm, out, raw, lidx, ones, zbuf, acc):
        c = lax.axis_index("c")
        s = lax.axis_index("s")

        def fill(i, _):
            zbuf[i, :] = jnp.zeros((L,), jnp.float32)
            ones[i, :] = jnp.ones((L,), jnp.float32)
            return 0
        lax.fori_loop(0, ZR, fill, 0)

        def fill2(i, _):
            ones[i, :] = jnp.ones((L,), jnp.float32)
            return 0
        lax.fori_loop(ZR, HCH, fill2, 0)

        for kk in range(TPR // ZR):
            pltpu.sync_copy(zbuf, acc.at[pl.ds(s * TPR + kk * ZR, ZR)])
        plsc.subcore_barrier()

        lane = lax.iota(jnp.int32, L)

        def chunk(g, _):
            base = g * HCH
            pltpu.sync_copy(idx_hbm.at[c * NS + s].at[pl.ds(base, HCH)], raw)

            def loc(j, _):
                v = raw[pl.ds(j * L, L)]
                pos = base + j * L + lane
                lidx[pl.ds(j * L, L)] = jnp.where(pos < EPT, v, N)
                return 0
            lax.fori_loop(0, HCH // L, loc, 0)
            pltpu.sync_copy(ones, acc.at[lidx], add=True)
            return 0
        lax.fori_loop(0, HNCHUNK, chunk, 0)
        plsc.subcore_barrier()

        for kk in range(TPR // ZR):
            off = s * TPR + kk * ZR
            pltpu.sync_copy(acc.at[pl.ds(off, ZR)], zbuf)
            pltpu.sync_copy(zbuf, out.at[c].at[pl.ds(off, ZR)])

    return k(idx2)


def _sc_scatter(mm, row2, col2):
    out_ty = jax.ShapeDtypeStruct((NC, N_PAD, W), jnp.float32)
    scratch = [
        pltpu.VMEM((CH,), jnp.int32), pltpu.VMEM((CH,), jnp.int32),
        pltpu.VMEM((CH,), jnp.int32),
        pltpu.VMEM((CH,), jnp.int32), pltpu.VMEM((CH,), jnp.int32),
        pltpu.VMEM((CH,), jnp.int32),
        pltpu.VMEM((CH, W), jnp.float32),
        pltpu.VMEM((CH, W), jnp.float32),
        pltpu.VMEM((ZR, W), jnp.float32),
        pltpu.VMEM_SHARED((N_PAD, W), jnp.float32),
        pltpu.SemaphoreType.DMA, pltpu.SemaphoreType.DMA,
        pltpu.SemaphoreType.DMA, pltpu.SemaphoreType.DMA,
        pltpu.SemaphoreType.DMA,
    ]

    @functools.partial(pl.kernel, out_type=out_ty, mesh=_mesh(),
                       scratch_types=scratch)
    def k(mm_hbm, row_hbm, col_hbm, out, gi0, gi1, gi2, li0, li1, li2,
          r0, r1, zbuf, acc, sg0, sg1, si0, si1, si2):
        c = lax.axis_index("c")
        s = lax.axis_index("s")

        def fz(i, _):
            def fz2(l, _):
                zbuf[i, pl.ds(l * L, L)] = jnp.zeros((L,), jnp.float32)
                return 0
            lax.fori_loop(0, W // L, fz2, 0)
            return 0
        lax.fori_loop(0, ZR, fz, 0)

        for kk in range(TPR // ZR):
            pltpu.sync_copy(zbuf, acc.at[pl.ds(s * TPR + kk * ZR, ZR)])
        plsc.subcore_barrier()

        lane = lax.iota(jnp.int32, L)
        LAST = (NCHUNK - 1) * CH
        isl = [(gi0, li0, si0), (gi1, li1, si1), (gi2, li2, si2)]
        rsl = [(r0, sg0), (r1, sg1)]

        def load_idx(g, gidx, lidx, semi):
            base = jnp.minimum(g * CH, LAST)
            pltpu.async_copy(row_hbm.at[s].at[pl.ds(base, CH)], gidx, semi)
            pltpu.async_copy(col_hbm.at[s].at[pl.ds(base, CH)], lidx, semi)

        def wait_idx(gidx, lidx, semi):
            pltpu.make_async_copy(
                row_hbm.at[s].at[pl.ds(0, CH)], gidx, semi).wait()
            pltpu.make_async_copy(
                col_hbm.at[s].at[pl.ds(0, CH)], lidx, semi).wait()

        def fire(g, gidx, lidx, rows, sem):
            def dbl(j, _):
                gidx[pl.ds(j * L, L)] = gidx[pl.ds(j * L, L)] * 2 + c
                return 0
            lax.fori_loop(0, CH // L, dbl, 0)
            pltpu.async_copy(mm_hbm.at[gidx], rows, sem)
            base = g * CH

            def loc(j, _):
                v = lidx[pl.ds(j * L, L)]
                pos = base + j * L + lane
                lidx[pl.ds(j * L, L)] = jnp.where(pos < EPT, v, N)
                return 0
            lax.fori_loop(0, CH // L, loc, 0)

        def scatter(gidx, lidx, rows, sem):
            pltpu.make_async_copy(mm_hbm.at[gidx], rows, sem).wait()
            pltpu.sync_copy(rows, acc.at[lidx], add=True)

        load_idx(0, *isl[0])
        load_idx(1, *isl[1])

        def block(b, _):
            for kk in range(6):
                g = 6 * b + kk
                gidx, lidx, si = isl[kk % 3]
                rows, sg = rsl[kk % 2]
                pgidx, plidx, _psi = isl[(kk + 2) % 3]
                prows, psg = rsl[(kk + 1) % 2]
                wait_idx(gidx, lidx, si)
                fire(g, gidx, lidx, rows, sg)
                if kk == 0:
                    @pl.when(b > 0)
                    def _(pgidx=pgidx, plidx=plidx, prows=prows, psg=psg):
                        scatter(pgidx, plidx, prows, psg)
                else:
                    scatter(pgidx, plidx, prows, psg)
                load_idx(g + 2, *isl[(kk + 2) % 3])
            return 0
        lax.fori_loop(0, NCHUNK // 6, block, 0)
        scatter(gi2, li2, r1, sg1)
        wait_idx(gi0, li0, si0)
        wait_idx(gi1, li1, si1)
        plsc.subcore_barrier()

        for kk in range(TPR // ZR):
            off = s * TPR + kk * ZR
            pltpu.sync_copy(acc.at[pl.ds(off, ZR)], zbuf)
            pltpu.sync_copy(zbuf, out.at[c].at[pl.ds(off, ZR)])

    return k(mm.reshape(N_PAD * 2, W), row2, col2)


def _tc_stage1(feats_p, W1p, b1r, Wg1, ch_p):
    def body(f_ref, w1_ref, b1_ref, wg_ref, ch_ref, out_ref):
        h1 = jnp.dot(f_ref[...], w1_ref[...],
                     preferred_element_type=jnp.float32) + b1_ref[...]
        dinv = lax.rsqrt(ch_ref[...] + 1.0)
        out_ref[...] = jnp.dot(h1, wg_ref[...],
                               preferred_element_type=jnp.float32) * dinv

    return pl.pallas_call(
        body,
        grid=(NB,),
        in_specs=[
            pl.BlockSpec((BN, KP), lambda i: (i, 0)),
            pl.BlockSpec((KP, D), lambda i: (0, 0)),
            pl.BlockSpec((1, D), lambda i: (0, 0)),
            pl.BlockSpec((D, D), lambda i: (0, 0)),
            pl.BlockSpec((BN, 1), lambda i: (i, 0)),
        ],
        out_specs=pl.BlockSpec((BN, D), lambda i: (i, 0)),
        out_shape=jax.ShapeDtypeStruct((N_PAD, D), jnp.float32),
    )(feats_p, W1p, b1r, Wg1, ch_p)


def _tc_stage2(Sa, Sb, mm1, ch_p, bgr, Wg):
    def body(sa_ref, sb_ref, m_ref, ch_ref, b_ref, w_ref, out_ref):
        dinv = lax.rsqrt(ch_ref[...] + 1.0)
        S = jnp.concatenate([sa_ref[...], sb_ref[...]], axis=1)
        h = jnp.maximum((S + m_ref[...]) * dinv + b_ref[...], 0.0)
        out_ref[...] = jnp.dot(h, w_ref[...],
                               preferred_element_type=jnp.float32) * dinv

    return pl.pallas_call(
        body,
        grid=(NB,),
        in_specs=[
            pl.BlockSpec((BN, W), lambda i: (i, 0)),
            pl.BlockSpec((BN, W), lambda i: (i, 0)),
            pl.BlockSpec((BN, D), lambda i: (i, 0)),
            pl.BlockSpec((BN, 1), lambda i: (i, 0)),
            pl.BlockSpec((1, D), lambda i: (0, 0)),
            pl.BlockSpec((D, D), lambda i: (0, 0)),
        ],
        out_specs=pl.BlockSpec((BN, D), lambda i: (i, 0)),
        out_shape=jax.ShapeDtypeStruct((N_PAD, D), jnp.float32),
    )(Sa, Sb, mm1, ch_p, bgr, Wg)


def _tc_final(Sa, Sb, mm2, ch_p, bgr, nn_col, Wf1p, bf1p, Wf2p, bf2p):
    def body(sa_ref, sb_ref, m_ref, ch_ref, b_ref, nn_ref, wf1_ref, bf1_ref,
             wf2_ref, bf2_ref, out_ref, acc):
        i = pl.program_id(0)

        @pl.when(i == 0)
        def _():
            acc[...] = jnp.zeros_like(acc)

        dinv = lax.rsqrt(ch_ref[...] + 1.0)
        S = jnp.concatenate([sa_ref[...], sb_ref[...]], axis=1)
        h = jnp.maximum((S + m_ref[...]) * dinv + b_ref[...], 0.0)
        nn = nn_ref[...]
        tri = (lax.broadcasted_iota(jnp.int32, (GP, GP), 1)
               <= lax.broadcasted_iota(jnp.int32, (GP, GP), 0)
               ).astype(jnp.float32)
        cum = jnp.dot(tri, nn, preferred_element_type=jnp.float32)
        start = cum - nn
        ig = (i * BN
              + lax.broadcasted_iota(jnp.int32, (1, BN), 1)).astype(
                  jnp.float32)
        ind = ((ig < cum) & (ig >= start)).astype(jnp.float32)
        acc[...] += jnp.dot(ind, h, preferred_element_type=jnp.float32)

        @pl.when(i == NB - 1)
        def _():
            a = jnp.maximum(
                jnp.dot(acc[...], wf1_ref[...],
                        preferred_element_type=jnp.float32) + bf1_ref[...],
                0.0)
            out_ref[...] = jnp.dot(
                a, wf2_ref[...],
                preferred_element_type=jnp.float32) + bf2_ref[...]

    return pl.pallas_call(
        body,
        grid=(NB,),
        in_specs=[
            pl.BlockSpec((BN, W), lambda i: (i, 0)),
            pl.BlockSpec((BN, W), lambda i: (i, 0)),
            pl.BlockSpec((BN, D), lambda i: (i, 0)),
            pl.BlockSpec((BN, 1), lambda i: (i, 0)),
            pl.BlockSpec((1, D), lambda i: (0, 0)),
            pl.BlockSpec((GP, 1), lambda i: (0, 0)),
            pl.BlockSpec((D, GP), lambda i: (0, 0)),
            pl.BlockSpec((1, GP), lambda i: (0, 0)),
            pl.BlockSpec((GP, 1), lambda i: (0, 0)),
            pl.BlockSpec((1, 1), lambda i: (0, 0)),
        ],
        out_specs=pl.BlockSpec((GP, 1), lambda i: (0, 0)),
        out_shape=jax.ShapeDtypeStruct((GP, 1), jnp.float32),
        scratch_shapes=[pltpu.VMEM((GP, D), jnp.float32)],
    )(Sa, Sb, mm2, ch_p, bgr, nn_col, Wf1p, bf1p, Wf2p, bf2p)


def _dbg_scatter(mm, row2, col2):
    mm2 = mm.reshape(N_PAD * 2, W)
    cidx = jnp.where(
        (jnp.arange(EPT_PAD)[None, :] < EPT), col2, N).reshape(-1)
    out = jnp.zeros((NC, N_PAD, W), jnp.float32)
    for c in range(NC):
        g = (row2 * 2 + c).reshape(-1)
        out = out.at[c, cidx].add(mm2[g])
    return out


def kernel(x_width, x, edge_index, node_num, W1, b1, Wg1, bg1, Wg2, bg2,
           Wf1, bf1, Wf2, bf2):
    row = edge_index[0]
    col = edge_index[1]
    row2 = jnp.pad(row.reshape(NS, EPT), ((0, 0), (0, EPT_PAD - EPT)))
    col2 = jnp.pad(col.reshape(NS, EPT), ((0, 0), (0, EPT_PAD - EPT)))
    rowh = jnp.pad(row.reshape(NS, EPT), ((0, 0), (0, HEPT_PAD - EPT)))
    colh = jnp.pad(col.reshape(NS, EPT), ((0, 0), (0, HEPT_PAD - EPT)))

    hist = _sc_hist(jnp.concatenate([rowh, colh], axis=0))
    nc_counts = hist[0, :N, 0]
    ch_p = hist[1, :, :1]

    feats = jnp.concatenate(
        [x_width[:, None], x, nc_counts[:, None]], axis=1)
    feats_p = jnp.pad(feats, ((0, N_PAD - N), (0, KP - 41)))
    W1p = jnp.pad(W1, ((0, KP - 41), (0, 0)))

    mm1 = _tc_stage1(feats_p, W1p, b1.reshape(1, D), Wg1, ch_p)

    s1 = _sc_scatter(mm1, row2, col2)
    mm2 = _tc_stage2(s1[0], s1[1], mm1, ch_p, bg1.reshape(1, D), Wg2)

    s2 = _sc_scatter(mm2, row2, col2)

    nn_col = jnp.pad(node_num.astype(jnp.float32),
                     (0, GP - G)).reshape(GP, 1)
    Wf1p = jnp.pad(Wf1, ((0, 0), (0, GP - G)))
    bf1p = jnp.pad(bf1, (0, GP - G)).reshape(1, GP)
    Wf2p = jnp.pad(Wf2, ((0, GP - G), (0, 0)))
    bf2p = bf2.reshape(1, 1)

    out = _tc_final(s2[0], s2[1], mm2, ch_p, bg2.reshape(1, D), nn_col,
                    Wf1p, bf1p, Wf2p, bf2p)
    return out[:G, 0]


# speedup vs baseline: 1.7975x; 1.7975x over previous
import functools

import jax
import jax.numpy as jnp
from jax import lax
from jax.experimental import pallas as pl
from jax.experimental.pallas import tpu as pltpu
from jax.experimental.pallas import tpu_sc as plsc

N = 10000
E = 160000
D = 256
G = 50

NC = 2
NS = 16
L = 16

W = D // NC
EPT = E // NS
CH = 128
NCHUNK = 80
EPT_PAD = NCHUNK * CH
HCH = 128
HNCHUNK = 80
HEPT_PAD = HNCHUNK * HCH
ZR = 32

N_PAD = 10240
TPR = N_PAD // NS
BN = 512
NB = N_PAD // BN
GP = 64
KP = 48


def _mesh():
    return plsc.VectorSubcoreMesh(core_axis_name="c", subcore_axis_name="s",
                                  num_cores=NC, num_subcores=NS)


def _sc_hist(idx2):
    out_ty = jax.ShapeDtypeStruct((NC, N_PAD, L), jnp.float32)
    scratch = [
        pltpu.VMEM((HCH,), jnp.int32),
        pltpu.VMEM((HCH,), jnp.int32),
        pltpu.VMEM((HCH, L), jnp.float32),
        pltpu.VMEM((ZR, L), jnp.float32),
        pltpu.VMEM_SHARED((N_PAD, L), jnp.float32),
    ]

    @functools.partial(pl.kernel, out_type=out_ty, mesh=_mesh(),
                       scratch_types=scratch)
    def k(idx_hbm, out, raw, lidx, ones, zbuf, acc):
        c = lax.axis_index("c")
        s = lax.axis_index("s")

        def fill(i, _):
            zbuf[i, :] = jnp.zeros((L,), jnp.float32)
            ones[i, :] = jnp.ones((L,), jnp.float32)
            return 0
        lax.fori_loop(0, ZR, fill, 0)

        def fill2(i, _):
            ones[i, :] = jnp.ones((L,), jnp.float32)
            return 0
        lax.fori_loop(ZR, HCH, fill2, 0)

        for kk in range(TPR // ZR):
            pltpu.sync_copy(zbuf, acc.at[pl.ds(s * TPR + kk * ZR, ZR)])
        plsc.subcore_barrier()

        lane = lax.iota(jnp.int32, L)

        def chunk(g, _):
            base = g * HCH
            pltpu.sync_copy(idx_hbm.at[c * NS + s].at[pl.ds(base, HCH)], raw)

            def loc(j, _):
                v = raw[pl.ds(j * L, L)]
                pos = base + j * L + lane
                lidx[pl.ds(j * L, L)] = jnp.where(pos < EPT, v, N)
                return 0
            lax.fori_loop(0, HCH // L, loc, 0)
            pltpu.sync_copy(ones, acc.at[lidx], add=True)
            return 0
        lax.fori_loop(0, HNCHUNK, chunk, 0)
        plsc.subcore_barrier()

        for kk in range(TPR // ZR):
            off = s * TPR + kk * ZR
            pltpu.sync_copy(acc.at[pl.ds(off, ZR)], zbuf)
            pltpu.sync_copy(zbuf, out.at[c].at[pl.ds(off, ZR)])

    return k(idx2)


def _sc_scatter(mm, row2, col2):
    out_ty = jax.ShapeDtypeStruct((NC, N_PAD, W), jnp.float32)
    scratch = [
        pltpu.VMEM((CH,), jnp.int32),
        pltpu.VMEM((CH,), jnp.int32),
        pltpu.VMEM((CH,), jnp.int32),
        pltpu.VMEM((CH,), jnp.int32),
        pltpu.VMEM((CH, W), jnp.float32),
        pltpu.VMEM((CH, W), jnp.float32),
        pltpu.VMEM((ZR, W), jnp.float32),
        pltpu.VMEM_SHARED((N_PAD, W), jnp.float32),
        pltpu.SemaphoreType.DMA,
        pltpu.SemaphoreType.DMA,
        pltpu.SemaphoreType.DMA,
    ]

    @functools.partial(pl.kernel, out_type=out_ty, mesh=_mesh(),
                       scratch_types=scratch)
    def k(mm_hbm, row_hbm, col_hbm, out, gidxa, gidxb, lidxa, lidxb,
          rowsa, rowsb, zbuf, acc, sema, semb, semi):
        c = lax.axis_index("c")
        s = lax.axis_index("s")

        def fz(i, _):
            def fz2(l, _):
                zbuf[i, pl.ds(l * L, L)] = jnp.zeros((L,), jnp.float32)
                return 0
            lax.fori_loop(0, W // L, fz2, 0)
            return 0
        lax.fori_loop(0, ZR, fz, 0)

        for kk in range(TPR // ZR):
            pltpu.sync_copy(zbuf, acc.at[pl.ds(s * TPR + kk * ZR, ZR)])
        plsc.subcore_barrier()

        lane = lax.iota(jnp.int32, L)

        def stage(g, gidx, lidx, rows, sem):
            base = g * CH
            pltpu.async_copy(row_hbm.at[s].at[pl.ds(base, CH)], gidx, semi)
            pltpu.async_copy(col_hbm.at[s].at[pl.ds(base, CH)], lidx, semi)
            pltpu.make_async_copy(
                row_hbm.at[s].at[pl.ds(0, CH)], gidx, semi).wait()
            pltpu.make_async_copy(
                col_hbm.at[s].at[pl.ds(0, CH)], lidx, semi).wait()

            def dbl(j, _):
                gidx[pl.ds(j * L, L)] = gidx[pl.ds(j * L, L)] * 2 + c
                return 0
            lax.fori_loop(0, CH // L, dbl, 0)
            gd = pltpu.async_copy(mm_hbm.at[gidx], rows, sem)

            def loc(j, _):
                v = lidx[pl.ds(j * L, L)]
                pos = base + j * L + lane
                lidx[pl.ds(j * L, L)] = jnp.where(pos < EPT, v, N)
                return 0
            lax.fori_loop(0, CH // L, loc, 0)
            return gd

        def scatter_b():
            pltpu.make_async_copy(mm_hbm.at[gidxb], rowsb, semb).wait()
            pltpu.sync_copy(rowsb, acc.at[lidxb], add=True)

        def pair(i, _):
            ga = stage(2 * i, gidxa, lidxa, rowsa, sema)

            @pl.when(i > 0)
            def _():
                scatter_b()

            stage(2 * i + 1, gidxb, lidxb, rowsb, semb)
            ga.wait()
            pltpu.sync_copy(rowsa, acc.at[lidxa], add=True)
            return 0
        lax.fori_loop(0, NCHUNK // 2, pair, 0)
        scatter_b()
        plsc.subcore_barrier()

        for kk in range(TPR // ZR):
            off = s * TPR + kk * ZR
            pltpu.sync_copy(acc.at[pl.ds(off, ZR)], zbuf)
            pltpu.sync_copy(zbuf, out.at[c].at[pl.ds(off, ZR)])

    return k(mm.reshape(N_PAD * 2, W), row2, col2)


def _tc_stage1(feats_p, W1p, b1r, Wg1, ch_p):
    def body(f_ref, w1_ref, b1_ref, wg_ref, ch_ref, out_ref):
        h1 = jnp.dot(f_ref[...], w1_ref[...],
                     preferred_element_type=jnp.float32) + b1_ref[...]
        dinv = lax.rsqrt(ch_ref[...] + 1.0)
        out_ref[...] = jnp.dot(h1, wg_ref[...],
                               preferred_element_type=jnp.float32) * dinv

    return pl.pallas_call(
        body,
        grid=(NB,),
        in_specs=[
            pl.BlockSpec((BN, KP), lambda i: (i, 0)),
            pl.BlockSpec((KP, D), lambda i: (0, 0)),
            pl.BlockSpec((1, D), lambda i: (0, 0)),
            pl.BlockSpec((D, D), lambda i: (0, 0)),
            pl.BlockSpec((BN, 1), lambda i: (i, 0)),
        ],
        out_specs=pl.BlockSpec((BN, D), lambda i: (i, 0)),
        out_shape=jax.ShapeDtypeStruct((N_PAD, D), jnp.float32),
    )(feats_p, W1p, b1r, Wg1, ch_p)


def _tc_stage2(Sa, Sb, mm1, ch_p, bgr, Wg):
    def body(sa_ref, sb_ref, m_ref, ch_ref, b_ref, w_ref, out_ref):
        dinv = lax.rsqrt(ch_ref[...] + 1.0)
        S = jnp.concatenate([sa_ref[...], sb_ref[...]], axis=1)
        h = jnp.maximum((S + m_ref[...]) * dinv + b_ref[...], 0.0)
        out_ref[...] = jnp.dot(h, w_ref[...],
                               preferred_element_type=jnp.float32) * dinv

    return pl.pallas_call(
        body,
        grid=(NB,),
        in_specs=[
            pl.BlockSpec((BN, W), lambda i: (i, 0)),
            pl.BlockSpec((BN, W), lambda i: (i, 0)),
            pl.BlockSpec((BN, D), lambda i: (i, 0)),
            pl.BlockSpec((BN, 1), lambda i: (i, 0)),
            pl.BlockSpec((1, D), lambda i: (0, 0)),
            pl.BlockSpec((D, D), lambda i: (0, 0)),
        ],
        out_specs=pl.BlockSpec((BN, D), lambda i: (i, 0)),
        out_shape=jax.ShapeDtypeStruct((N_PAD, D), jnp.float32),
    )(Sa, Sb, mm1, ch_p, bgr, Wg)


def _tc_final(Sa, Sb, mm2, ch_p, bgr, nn_col, Wf1p, bf1p, Wf2p, bf2p):
    def body(sa_ref, sb_ref, m_ref, ch_ref, b_ref, nn_ref, wf1_ref, bf1_ref,
             wf2_ref, bf2_ref, out_ref, acc):
        i = pl.program_id(0)

        @pl.when(i == 0)
        def _():
            acc[...] = jnp.zeros_like(acc)

        dinv = lax.rsqrt(ch_ref[...] + 1.0)
        S = jnp.concatenate([sa_ref[...], sb_ref[...]], axis=1)
        h = jnp.maximum((S + m_ref[...]) * dinv + b_ref[...], 0.0)
        nn = nn_ref[...]
        tri = (lax.broadcasted_iota(jnp.int32, (GP, GP), 1)
               <= lax.broadcasted_iota(jnp.int32, (GP, GP), 0)
               ).astype(jnp.float32)
        cum = jnp.dot(tri, nn, preferred_element_type=jnp.float32)
        start = cum - nn
        ig = (i * BN
              + lax.broadcasted_iota(jnp.int32, (1, BN), 1)).astype(
                  jnp.float32)
        ind = ((ig < cum) & (ig >= start)).astype(jnp.float32)
        acc[...] += jnp.dot(ind, h, preferred_element_type=jnp.float32)

        @pl.when(i == NB - 1)
        def _():
            a = jnp.maximum(
                jnp.dot(acc[...], wf1_ref[...],
                        preferred_element_type=jnp.float32) + bf1_ref[...],
                0.0)
            out_ref[...] = jnp.dot(
                a, wf2_ref[...],
                preferred_element_type=jnp.float32) + bf2_ref[...]

    return pl.pallas_call(
        body,
        grid=(NB,),
        in_specs=[
            pl.BlockSpec((BN, W), lambda i: (i, 0)),
            pl.BlockSpec((BN, W), lambda i: (i, 0)),
            pl.BlockSpec((BN, D), lambda i: (i, 0)),
            pl.BlockSpec((BN, 1), lambda i: (i, 0)),
            pl.BlockSpec((1, D), lambda i: (0, 0)),
            pl.BlockSpec((GP, 1), lambda i: (0, 0)),
            pl.BlockSpec((D, GP), lambda i: (0, 0)),
            pl.BlockSpec((1, GP), lambda i: (0, 0)),
            pl.BlockSpec((GP, 1), lambda i: (0, 0)),
            pl.BlockSpec((1, 1), lambda i: (0, 0)),
        ],
        out_specs=pl.BlockSpec((GP, 1), lambda i: (0, 0)),
        out_shape=jax.ShapeDtypeStruct((GP, 1), jnp.float32),
        scratch_shapes=[pltpu.VMEM((GP, D), jnp.float32)],
    )(Sa, Sb, mm2, ch_p, bgr, nn_col, Wf1p, bf1p, Wf2p, bf2p)


def _dbg_scatter(mm, row2, col2):
    mm2 = mm.reshape(N_PAD * 2, W)
    cidx = jnp.where(
        (jnp.arange(EPT_PAD)[None, :] < EPT), col2, N).reshape(-1)
    out = jnp.zeros((NC, N_PAD, W), jnp.float32)
    for c in range(NC):
        g = (row2 * 2 + c).reshape(-1)
        out = out.at[c, cidx].add(mm2[g])
    return out


def kernel(x_width, x, edge_index, node_num, W1, b1, Wg1, bg1, Wg2, bg2,
           Wf1, bf1, Wf2, bf2):
    row = edge_index[0]
    col = edge_index[1]
    row2 = jnp.pad(row.reshape(NS, EPT), ((0, 0), (0, EPT_PAD - EPT)))
    col2 = jnp.pad(col.reshape(NS, EPT), ((0, 0), (0, EPT_PAD - EPT)))
    rowh = jnp.pad(row.reshape(NS, EPT), ((0, 0), (0, HEPT_PAD - EPT)))
    colh = jnp.pad(col.reshape(NS, EPT), ((0, 0), (0, HEPT_PAD - EPT)))

    hist = _sc_hist(jnp.concatenate([rowh, colh], axis=0))
    nc_counts = hist[0, :N, 0]
    ch_p = hist[1, :, :1]

    feats = jnp.concatenate(
        [x_width[:, None], x, nc_counts[:, None]], axis=1)
    feats_p = jnp.pad(feats, ((0, N_PAD - N), (0, KP - 41)))
    W1p = jnp.pad(W1, ((0, KP - 41), (0, 0)))

    mm1 = _tc_stage1(feats_p, W1p, b1.reshape(1, D), Wg1, ch_p)

    s1 = _sc_scatter(mm1, row2, col2)
    mm2 = _tc_stage2(s1[0], s1[1], mm1, ch_p, bg1.reshape(1, D), Wg2)

    s2 = _sc_scatter(mm2, row2, col2)

    nn_col = jnp.pad(node_num.astype(jnp.float32),
                     (0, GP - G)).reshape(GP, 1)
    Wf1p = jnp.pad(Wf1, ((0, 0), (0, GP - G)))
    bf1p = jnp.pad(bf1, (0, GP - G)).reshape(1, GP)
    Wf2p = jnp.pad(Wf2, ((0, GP - G), (0, 0)))
    bf2p = bf2.reshape(1, 1)

    out = _tc_final(s2[0], s2[1], mm2, ch_p, bg2.reshape(1, D), nn_col,
                    Wf1p, bf1p, Wf2p, bf2p)
    return out[:G, 0]
